# baseline (speedup 1.0000x reference)
import jax
import jax.numpy as jnp
from jax import lax
from jax.experimental import pallas as pl
from jax.experimental.pallas import tpu as pltpu
from jax.experimental.pallas import tpu_sc as plsc

B = 16384
D = 64
VOCABS = (235, 2, 111, 526, 21, 50)
NT = len(VOCABS)
OUT_W = NT * D + 1

_info = plsc.get_sparse_core_info()
_NC, _NS = _info.num_cores, _info.num_subcores
NW = _NC * _NS
B_PER_W = B // NW
CHUNK = 64
N_CHUNKS = B_PER_W // CHUNK
NBUF = 2


def _body(idx_all, age8,
          t_u, t_gd, t_oc, t_zc, t_m, t_g,
          out, idx_v, rows_v, age_v,
          l_u, l_gd, l_oc, l_zc, l_m, l_g,
          sem_i, sem_g, sem_w, sem_a, sem_t):
    sid = lax.axis_index("s")
    wid = sid * _NC + lax.axis_index("c")
    base = wid * B_PER_W
    tbl_hbm = (t_u, t_gd, t_oc, t_zc, t_m, t_g)
    tbl_loc = (l_u, l_gd, l_oc, l_zc, l_m, l_g)

    @pl.when(sid == 0)
    def _stage():
        stage = [pltpu.async_copy(tbl_hbm[t], tbl_loc[t], sem_t)
                 for t in range(NT)]
        for c in stage:
            c.wait()

    age_in = pltpu.async_copy(age8.at[pl.ds(base, B_PER_W), pl.ds(0, 1)],
                              age_v, sem_a)

    def start_idx(k):
        r = base + k * CHUNK
        return pltpu.async_copy(idx_all.at[:, pl.ds(r, CHUNK)],
                                idx_v.at[k % NBUF], sem_i)

    idx_copies = [start_idx(0)]
    plsc.subcore_barrier()
    write_copies = []
    for k in range(N_CHUNKS):
        b = k % NBUF
        r = base + k * CHUNK
        idx_copies[k].wait()
        if k + 1 < N_CHUNKS:
            idx_copies.append(start_idx(k + 1))
        if k >= NBUF:
            for c in write_copies[k - NBUF]:
                c.wait()
        gathers = [
            pltpu.async_copy(tbl_loc[t].at[idx_v.at[b, t]],
                             rows_v.at[b, t], sem_g)
            for t in range(NT)
        ]
        for c in gathers:
            c.wait()
        write_copies.append([
            pltpu.async_copy(rows_v.at[b, t],
                             out.at[pl.ds(r, CHUNK), pl.ds(t * D, D)], sem_w)
            for t in range(NT)
        ])
    age_in.wait()
    age_out = pltpu.async_copy(
        age_v, out.at[pl.ds(base, B_PER_W), pl.ds(NT * D, 1)], sem_a)
    for cs in write_copies[-NBUF:]:
        for c in cs:
            c.wait()
    age_out.wait()


def kernel(user_id, movie_id, genres, gender, age, occupation, zip_code,
           user_emb, gender_emb, occupation_emb, zip_code_emb, movie_emb,
           genres_emb):
    idx_all = jnp.stack([
        user_id.astype(jnp.int32), gender.astype(jnp.int32),
        occupation.astype(jnp.int32), zip_code.astype(jnp.int32),
        movie_id.astype(jnp.int32), genres.astype(jnp.int32)])
    age8 = jnp.pad(age, ((0, 0), (0, 7)))
    mesh = plsc.VectorSubcoreMesh(core_axis_name="c", subcore_axis_name="s")
    k = pl.kernel(
        _body,
        out_type=jax.ShapeDtypeStruct((B, OUT_W), jnp.float32),
        mesh=mesh,
        scratch_types=[
            pltpu.VMEM((NBUF, NT, CHUNK), jnp.int32),
            pltpu.VMEM((NBUF, NT, CHUNK, D), jnp.float32),
            pltpu.VMEM((B_PER_W, 1), jnp.float32),
        ] + [pltpu.VMEM_SHARED((v, D), jnp.float32) for v in VOCABS] + [
            pltpu.SemaphoreType.DMA,
            pltpu.SemaphoreType.DMA,
            pltpu.SemaphoreType.DMA,
            pltpu.SemaphoreType.DMA,
            pltpu.SemaphoreType.DMA,
        ],
        compiler_params=pltpu.CompilerParams(use_tc_tiling_on_sc=False),
    )
    return k(idx_all, age8,
             user_emb, gender_emb, occupation_emb, zip_code_emb, movie_emb,
             genres_emb)


# speedup vs baseline: 1.0290x; 1.0290x over previous
import jax
import jax.numpy as jnp
from jax import lax
from jax.experimental import pallas as pl
from jax.experimental.pallas import tpu as pltpu
from jax.experimental.pallas import tpu_sc as plsc

B = 16384
D = 64
VOCABS = (235, 2, 111, 526, 21, 50)
NT = len(VOCABS)
OUT_W = NT * D + 1

_info = plsc.get_sparse_core_info()
_NC, _NS = _info.num_cores, _info.num_subcores
NW = _NC * _NS
B_PER_W = B // NW
CHUNK = 128
N_CHUNKS = B_PER_W // CHUNK
NBUF = 2


def _body(u_i, gd_i, oc_i, zc_i, m_i, g_i, age,
          t_u, t_gd, t_oc, t_zc, t_m, t_g,
          out, idx_v, rows_v, age_v,
          l_u, l_gd, l_oc, l_zc, l_m, l_g,
          sem_i, sem_g, sem_w, sem_a, sem_t):
    sid = lax.axis_index("s")
    wid = sid * _NC + lax.axis_index("c")
    base = wid * B_PER_W
    idx_hbm = (u_i, gd_i, oc_i, zc_i, m_i, g_i)
    tbl_hbm = (t_u, t_gd, t_oc, t_zc, t_m, t_g)
    tbl_loc = (l_u, l_gd, l_oc, l_zc, l_m, l_g)

    @pl.when(sid == 0)
    def _stage():
        stage = [pltpu.async_copy(tbl_hbm[t], tbl_loc[t], sem_t)
                 for t in range(NT)]
        for c in stage:
            c.wait()

    age_in = pltpu.async_copy(age.at[pl.ds(base, B_PER_W)], age_v, sem_a)

    def start_idx(k):
        r = base + k * CHUNK
        return [pltpu.async_copy(idx_hbm[t].at[pl.ds(r, CHUNK)],
                                 idx_v.at[k % NBUF, t], sem_i)
                for t in range(NT)]

    idx_copies = [start_idx(0)]
    plsc.subcore_barrier()
    write_copies = []
    for k in range(N_CHUNKS):
        b = k % NBUF
        r = base + k * CHUNK
        for c in idx_copies[k]:
            c.wait()
        if k + 1 < N_CHUNKS:
            idx_copies.append(start_idx(k + 1))
        if k >= NBUF:
            for c in write_copies[k - NBUF]:
                c.wait()
        gathers = [
            pltpu.async_copy(tbl_loc[t].at[idx_v.at[b, t]],
                             rows_v.at[b, t], sem_g)
            for t in range(NT)
        ]
        for c in gathers:
            c.wait()
        write_copies.append([
            pltpu.async_copy(rows_v.at[b, t],
                             out.at[pl.ds(r, CHUNK), pl.ds(t * D, D)], sem_w)
            for t in range(NT)
        ])
    age_in.wait()
    age_out = pltpu.async_copy(
        age_v, out.at[pl.ds(base, B_PER_W), pl.ds(NT * D, 1)], sem_a)
    for cs in write_copies[-NBUF:]:
        for c in cs:
            c.wait()
    age_out.wait()


def kernel(user_id, movie_id, genres, gender, age, occupation, zip_code,
           user_emb, gender_emb, occupation_emb, zip_code_emb, movie_emb,
           genres_emb):
    mesh = plsc.VectorSubcoreMesh(core_axis_name="c", subcore_axis_name="s")
    k = pl.kernel(
        _body,
        out_type=jax.ShapeDtypeStruct((B, OUT_W), jnp.float32),
        mesh=mesh,
        scratch_types=[
            pltpu.VMEM((NBUF, NT, CHUNK), jnp.int32),
            pltpu.VMEM((NBUF, NT, CHUNK, D), jnp.float32),
            pltpu.VMEM((B_PER_W, 1), jnp.float32),
        ] + [pltpu.VMEM_SHARED((v, D), jnp.float32) for v in VOCABS] + [
            pltpu.SemaphoreType.DMA,
            pltpu.SemaphoreType.DMA,
            pltpu.SemaphoreType.DMA,
            pltpu.SemaphoreType.DMA,
            pltpu.SemaphoreType.DMA,
        ],
        compiler_params=pltpu.CompilerParams(use_tc_tiling_on_sc=False),
    )
    return k(user_id.astype(jnp.int32), gender.astype(jnp.int32),
             occupation.astype(jnp.int32), zip_code.astype(jnp.int32),
             movie_id.astype(jnp.int32), genres.astype(jnp.int32), age,
             user_emb, gender_emb, occupation_emb, zip_code_emb, movie_emb,
             genres_emb)


# speedup vs baseline: 1.0465x; 1.0170x over previous
import jax
import jax.numpy as jnp
from jax import lax
from jax.experimental import pallas as pl
from jax.experimental.pallas import tpu as pltpu
from jax.experimental.pallas import tpu_sc as plsc

B = 16384
D = 64
VOCABS = (235, 2, 111, 526, 21, 50)
NT = len(VOCABS)
OUT_W = NT * D + 1

_info = plsc.get_sparse_core_info()
_NC, _NS = _info.num_cores, _info.num_subcores
NW = _NC * _NS
B_PER_W = B // NW
CHUNK = 128
N_CHUNKS = B_PER_W // CHUNK
NBUF = 2


def _body(u_i, gd_i, oc_i, zc_i, m_i, g_i, age,
          t_u, t_gd, t_oc, t_zc, t_m, t_g,
          out, idx_v, rows_v, age_v,
          l_u, l_gd, l_oc, l_zc, l_m, l_g,
          sem_i, sem_g, sem_w, sem_a, sem_t):
    sid = lax.axis_index("s")
    wid = sid * _NC + lax.axis_index("c")
    base = wid * B_PER_W
    idx_hbm = (u_i, gd_i, oc_i, zc_i, m_i, g_i)
    tbl_hbm = (t_u, t_gd, t_oc, t_zc, t_m, t_g)
    tbl_loc = (l_u, l_gd, l_oc, l_zc, l_m, l_g)

    @pl.when(sid == 0)
    def _stage():
        stage = [pltpu.async_copy(tbl_hbm[t], tbl_loc[t], sem_t)
                 for t in range(NT)]
        for c in stage:
            c.wait()

    age_in = pltpu.async_copy(age.at[pl.ds(base, B_PER_W), pl.ds(0, 1)],
                              age_v, sem_a)

    def start_idx(k):
        r = base + k * CHUNK
        return [pltpu.async_copy(idx_hbm[t].at[pl.ds(r, CHUNK)],
                                 idx_v.at[k % NBUF, t], sem_i)
                for t in range(NT)]

    idx_copies = [start_idx(0)]
    plsc.subcore_barrier()
    write_copies = []
    for k in range(N_CHUNKS):
        b = k % NBUF
        r = base + k * CHUNK
        for c in idx_copies[k]:
            c.wait()
        if k + 1 < N_CHUNKS:
            idx_copies.append(start_idx(k + 1))
        if k >= NBUF:
            for c in write_copies[k - NBUF]:
                c.wait()
        gathers = [
            pltpu.async_copy(tbl_loc[t].at[idx_v.at[b, t]],
                             rows_v.at[b, t], sem_g)
            for t in range(NT)
        ]
        for c in gathers:
            c.wait()
        write_copies.append([
            pltpu.async_copy(rows_v.at[b, t],
                             out.at[pl.ds(r, CHUNK), pl.ds(t * D, D)], sem_w)
            for t in range(NT)
        ])
    age_in.wait()
    age_out = pltpu.async_copy(
        age_v, out.at[pl.ds(base, B_PER_W), pl.ds(NT * D, 1)], sem_a)
    for cs in write_copies[-NBUF:]:
        for c in cs:
            c.wait()
    age_out.wait()


def kernel(user_id, movie_id, genres, gender, age, occupation, zip_code,
           user_emb, gender_emb, occupation_emb, zip_code_emb, movie_emb,
           genres_emb):
    mesh = plsc.VectorSubcoreMesh(core_axis_name="c", subcore_axis_name="s")
    k = pl.kernel(
        _body,
        out_type=jax.ShapeDtypeStruct((B, OUT_W), jnp.float32),
        mesh=mesh,
        scratch_types=[
            pltpu.VMEM((NBUF, NT, CHUNK), jnp.int32),
            pltpu.VMEM((NBUF, NT, CHUNK, D), jnp.float32),
            pltpu.VMEM((B_PER_W, 1), jnp.float32),
        ] + [pltpu.VMEM_SHARED((v, D), jnp.float32) for v in VOCABS] + [
            pltpu.SemaphoreType.DMA,
            pltpu.SemaphoreType.DMA,
            pltpu.SemaphoreType.DMA,
            pltpu.SemaphoreType.DMA,
            pltpu.SemaphoreType.DMA,
        ],
        compiler_params=pltpu.CompilerParams(use_tc_tiling_on_sc=False),
    )
    return k(user_id.astype(jnp.int32), gender.astype(jnp.int32),
             occupation.astype(jnp.int32), zip_code.astype(jnp.int32),
             movie_id.astype(jnp.int32), genres.astype(jnp.int32),
             age * jnp.float32(1) * (jnp.arange(8) == 0),
             user_emb, gender_emb, occupation_emb, zip_code_emb, movie_emb,
             genres_emb)


# speedup vs baseline: 1.0656x; 1.0183x over previous
import jax
import jax.numpy as jnp
from jax import lax
from jax.experimental import pallas as pl
from jax.experimental.pallas import tpu as pltpu
from jax.experimental.pallas import tpu_sc as plsc

B = 16384
D = 64
VOCABS = (235, 2, 111, 526, 21, 50)
NT = len(VOCABS)
V_ALL = sum(VOCABS)
OFFS = tuple(sum(VOCABS[:t]) for t in range(NT))
OUT_W = NT * D + 1

_info = plsc.get_sparse_core_info()
_NC, _NS = _info.num_cores, _info.num_subcores
NW = _NC * _NS
B_PER_W = B // NW
CHUNK = 128
N_CHUNKS = B_PER_W // CHUNK
NBUF = 2


def _body(u_i, gd_i, oc_i, zc_i, m_i, g_i, age,
          tbl_hbm,
          out, idx_v, rows_v, age_v, tbl_loc,
          sem_i, sem_g, sem_w, sem_a, sem_t):
    sid = lax.axis_index("s")
    wid = sid * _NC + lax.axis_index("c")
    base = wid * B_PER_W
    idx_hbm = (u_i, gd_i, oc_i, zc_i, m_i, g_i)

    @pl.when(sid == 0)
    def _stage():
        pltpu.async_copy(tbl_hbm, tbl_loc, sem_t).wait()

    age_in = pltpu.async_copy(age.at[pl.ds(base, B_PER_W), pl.ds(0, 1)],
                              age_v, sem_a)

    def start_idx(k):
        r = base + k * CHUNK
        return [pltpu.async_copy(idx_hbm[t].at[pl.ds(r, CHUNK)],
                                 idx_v.at[k % NBUF, t], sem_i)
                for t in range(NT)]

    idx_copies = [start_idx(0)]
    plsc.subcore_barrier()
    write_copies = []
    for k in range(N_CHUNKS):
        b = k % NBUF
        r = base + k * CHUNK
        for c in idx_copies[k]:
            c.wait()
        if k + 1 < N_CHUNKS:
            idx_copies.append(start_idx(k + 1))
        if k >= NBUF:
            for c in write_copies[k - NBUF]:
                c.wait()
        for t in range(1, NT):
            for j in range(CHUNK // 16):
                sl = pl.ds(j * 16, 16)
                idx_v[b, t, sl] = idx_v[b, t, sl] + OFFS[t]
        gathers = [
            pltpu.async_copy(tbl_loc.at[idx_v.at[b, t]],
                             rows_v.at[b, t], sem_g)
            for t in range(NT)
        ]
        for c in gathers:
            c.wait()
        write_copies.append([
            pltpu.async_copy(rows_v.at[b, t],
                             out.at[pl.ds(r, CHUNK), pl.ds(t * D, D)], sem_w)
            for t in range(NT)
        ])
    age_in.wait()
    age_out = pltpu.async_copy(
        age_v, out.at[pl.ds(base, B_PER_W), pl.ds(NT * D, 1)], sem_a)
    for cs in write_copies[-NBUF:]:
        for c in cs:
            c.wait()
    age_out.wait()


def kernel(user_id, movie_id, genres, gender, age, occupation, zip_code,
           user_emb, gender_emb, occupation_emb, zip_code_emb, movie_emb,
           genres_emb):
    mesh = plsc.VectorSubcoreMesh(core_axis_name="c", subcore_axis_name="s")
    k = pl.kernel(
        _body,
        out_type=jax.ShapeDtypeStruct((B, OUT_W), jnp.float32),
        mesh=mesh,
        scratch_types=[
            pltpu.VMEM((NBUF, NT, CHUNK), jnp.int32),
            pltpu.VMEM((NBUF, NT, CHUNK, D), jnp.float32),
            pltpu.VMEM((B_PER_W, 1), jnp.float32),
            pltpu.VMEM_SHARED((V_ALL, D), jnp.float32),
        ] + [
            pltpu.SemaphoreType.DMA,
            pltpu.SemaphoreType.DMA,
            pltpu.SemaphoreType.DMA,
            pltpu.SemaphoreType.DMA,
            pltpu.SemaphoreType.DMA,
        ],
        compiler_params=pltpu.CompilerParams(use_tc_tiling_on_sc=False),
    )
    return k(user_id.astype(jnp.int32), gender.astype(jnp.int32),
             occupation.astype(jnp.int32), zip_code.astype(jnp.int32),
             movie_id.astype(jnp.int32), genres.astype(jnp.int32),
             age * jnp.float32(1) * (jnp.arange(8) == 0),
             jnp.concatenate([user_emb, gender_emb, occupation_emb,
                              zip_code_emb, movie_emb, genres_emb]))
